# baseline (speedup 1.0000x reference)
import functools

import jax
import jax.numpy as jnp
from jax import lax
from jax.experimental import pallas as pl
from jax.experimental.pallas import tpu as pltpu
from jax.experimental.pallas import tpu_sc as plsc

BATCH = 64
CHANNELS = 3
SPATIAL = 512
N = SPATIAL * SPATIAL
P = BATCH * CHANNELS
NUM_CORES = 2
NUM_SUBCORES = 16
CHUNK = N // NUM_SUBCORES
PP = P // NUM_CORES


def _shuffle(x2, perm):
    mesh = plsc.VectorSubcoreMesh(core_axis_name="c", subcore_axis_name="s")

    @functools.partial(
        pl.kernel,
        mesh=mesh,
        out_type=jax.ShapeDtypeStruct((P, N), jnp.float32),
        scratch_types=[
            pltpu.VMEM((CHUNK,), jnp.int32),
            pltpu.VMEM((CHUNK,), jnp.float32),
            pltpu.VMEM((CHUNK,), jnp.float32),
            pltpu.VMEM_SHARED((N,), jnp.float32),
            pltpu.VMEM_SHARED((N,), jnp.float32),
            pltpu.VMEM_SHARED((N,), jnp.float32),
            pltpu.VMEM_SHARED((N,), jnp.float32),
            pltpu.SemaphoreType.DMA,
            pltpu.SemaphoreType.DMA,
            pltpu.SemaphoreType.DMA,
            pltpu.SemaphoreType.DMA,
        ],
    )
    def k(x_hbm, perm_hbm, out_hbm, idx_v, out0, out1, sh0, sh1, sh2,
          dummy_sh, sem_d, sem_l, sem_g, sem_s):
        cid = lax.axis_index("c")
        sid = lax.axis_index("s")
        col0 = sid * CHUNK
        base = cid * PP
        pltpu.sync_copy(perm_hbm.at[pl.ds(col0, CHUNK)], idx_v)

        outs = (out0, out1)

        def load(p, sh):
            return pltpu.make_async_copy(
                x_hbm.at[p, pl.ds(col0, CHUNK)],
                sh.at[pl.ds(col0, CHUNK)],
                sem_l,
            )

        def gather(sh, out_v):
            return pltpu.make_async_copy(sh.at[idx_v], out_v, sem_g)

        def store(p, out_v):
            return pltpu.make_async_copy(
                out_v, out_hbm.at[p, pl.ds(col0, CHUNK)], sem_s
            )

        load(base, sh0).start()
        load(base + 1, sh1).start()
        load(base, sh0).wait()
        plsc.subcore_barrier()
        gather(sh0, out0).start()

        def step(i, sh_cur, sh_nxt, sh_nxt2, out_cur, out_nxt):
            del sh_cur
            @pl.when(i + 1 < base + PP)
            def _():
                load(i + 1, sh_nxt).wait()

            @pl.when(i >= base + 1)
            def _():
                store(i - 1, out_nxt).wait()

            plsc.subcore_barrier()

            @pl.when(i + 1 < base + PP)
            def _():
                gather(sh_nxt, out_nxt).start()

            @pl.when(i + 2 < base + PP)
            def _():
                load(i + 2, sh_nxt2).start()

            @pl.when(i >= base + 1)
            def _():
                pltpu.make_async_copy(
                    out_nxt, dummy_sh.at[idx_v], sem_d
                ).wait()

            pltpu.make_async_copy(out_nxt, dummy_sh.at[idx_v], sem_d).start()
            pltpu.make_async_copy(sh_nxt.at[idx_v], out_cur, sem_g).wait()
            store(i, out_cur).start()

        def body(j, _):
            i = base + 6 * j
            sh = (sh0, sh1, sh2)
            for u in range(6):
                step(
                    i + u,
                    sh[u % 3],
                    sh[(u + 1) % 3],
                    sh[(u + 2) % 3],
                    outs[u % 2],
                    outs[(u + 1) % 2],
                )
            return ()

        lax.fori_loop(0, PP // 6, body, ())
        pltpu.make_async_copy(out0, dummy_sh.at[idx_v], sem_d).wait()
        store(base, outs[(PP - 1) % 2]).wait()

    return k(x2, perm)


def kernel(x, permutation):
    x2 = x.reshape(P, N)
    perm = permutation.astype(jnp.int32)
    out = _shuffle(x2, perm)
    return out.reshape(x.shape)


# speedup vs baseline: 1.5024x; 1.5024x over previous
import functools

import jax
import jax.numpy as jnp
from jax import lax
from jax.experimental import pallas as pl
from jax.experimental.pallas import tpu as pltpu
from jax.experimental.pallas import tpu_sc as plsc

BATCH = 64
CHANNELS = 3
SPATIAL = 512
N = SPATIAL * SPATIAL
P = BATCH * CHANNELS
NUM_CORES = 2
NUM_SUBCORES = 16
CHUNK = N // NUM_SUBCORES
PP = P // NUM_CORES
PAIRS = PP // 2
VSTEPS = CHUNK // 16
UNROLL = 8


def _shuffle(x2, perm):
    mesh = plsc.VectorSubcoreMesh(core_axis_name="c", subcore_axis_name="s")

    @functools.partial(
        pl.kernel,
        mesh=mesh,
        out_type=jax.ShapeDtypeStruct((P, N), jnp.float32),
        compiler_params=pltpu.CompilerParams(needs_layout_passes=False),
        scratch_types=[
            pltpu.VMEM((CHUNK,), jnp.int32),
            pltpu.VMEM((CHUNK,), jnp.int32),
            pltpu.VMEM((CHUNK,), jnp.float32),
            pltpu.VMEM((CHUNK,), jnp.float32),
            pltpu.VMEM_SHARED((N,), jnp.float32),
            pltpu.VMEM_SHARED((N,), jnp.float32),
            pltpu.VMEM_SHARED((N,), jnp.float32),
            pltpu.SemaphoreType.DMA,
            pltpu.SemaphoreType.DMA,
            pltpu.SemaphoreType.DMA,
            pltpu.SemaphoreType.DMA,
            pltpu.SemaphoreType.DMA,
        ],
    )
    def k(x_hbm, perm_hbm, out_hbm, idx_v, iperm_v, gout, ssrc,
          inA, inB, outS, sem_l, sem_g, sem_sc, sem_st, sem_rb):
        cid = lax.axis_index("c")
        sid = lax.axis_index("s")
        col0 = sid * CHUNK
        base = cid * PP
        pltpu.sync_copy(perm_hbm.at[pl.ds(col0, CHUNK)], idx_v)

        def iota_body(kk, _):
            for u in range(UNROLL):
                off = (kk * UNROLL + u) * 16
                q = col0 + off + lax.iota(jnp.int32, 16)
                gout[pl.ds(off, 16)] = plsc.bitcast(q, jnp.float32)
            return ()

        lax.fori_loop(0, VSTEPS // UNROLL, iota_body, ())
        pltpu.async_copy(gout, outS.at[idx_v], sem_sc).wait()
        plsc.subcore_barrier()
        pltpu.sync_copy(outS.at[pl.ds(col0, CHUNK)], gout)

        def cast_body(kk, _):
            for u in range(UNROLL):
                off = (kk * UNROLL + u) * 16
                iperm_v[pl.ds(off, 16)] = plsc.bitcast(
                    gout[pl.ds(off, 16)], jnp.int32
                )
            return ()

        lax.fori_loop(0, VSTEPS // UNROLL, cast_body, ())

        def loadA(p, sl):
            return pltpu.make_async_copy(
                x_hbm.at[p, pl.ds(col0, CHUNK)],
                sl.at[pl.ds(col0, CHUNK)],
                sem_l,
            )

        def gath(sl):
            return pltpu.make_async_copy(sl.at[idx_v], gout, sem_g)

        def storeA(p):
            return pltpu.make_async_copy(
                gout, out_hbm.at[p, pl.ds(col0, CHUNK)], sem_st
            )

        def scat():
            return pltpu.make_async_copy(ssrc, outS.at[iperm_v], sem_sc)

        def readb(p):
            return pltpu.make_async_copy(
                outS.at[pl.ds(col0, CHUNK)],
                out_hbm.at[p, pl.ds(col0, CHUNK)],
                sem_rb,
            )

        loadA(base, inA).start()
        loadA(base, inA).wait()
        plsc.subcore_barrier()
        gath(inA).start()

        def pair(j, in_cur, in_nxt):
            a = base + 2 * j
            b = a + 1

            @pl.when(j >= 1)
            def _():
                readb(b - 2).wait()

            plsc.subcore_barrier()
            pltpu.sync_copy(x_hbm.at[b, pl.ds(col0, CHUNK)], ssrc)
            scat().start()

            @pl.when(j + 1 < PAIRS)
            def _():
                loadA(a + 2, in_nxt).start()

            gath(in_cur).wait()
            storeA(a).start()
            scat().wait()

            @pl.when(j + 1 < PAIRS)
            def _():
                loadA(a + 2, in_nxt).wait()

            plsc.subcore_barrier()
            readb(b).start()

            @pl.when(j + 1 < PAIRS)
            def _():
                storeA(a).wait()
                gath(in_nxt).start()

        def body(jj, _):
            pair(2 * jj, inA, inB)
            pair(2 * jj + 1, inB, inA)
            return ()

        lax.fori_loop(0, PAIRS // 2, body, ())
        readb(base).wait()
        storeA(base).wait()

    return k(x2, perm)


def kernel(x, permutation):
    x2 = x.reshape(P, N)
    perm = permutation.astype(jnp.int32)
    out = _shuffle(x2, perm)
    return out.reshape(x.shape)


# speedup vs baseline: 1.5480x; 1.0304x over previous
import functools

import jax
import jax.numpy as jnp
from jax import lax
from jax.experimental import pallas as pl
from jax.experimental.pallas import tpu as pltpu
from jax.experimental.pallas import tpu_sc as plsc

BATCH = 64
CHANNELS = 3
SPATIAL = 512
N = SPATIAL * SPATIAL
P = BATCH * CHANNELS
NUM_CORES = 2
NUM_SUBCORES = 16
CHUNK = N // NUM_SUBCORES
PP = P // NUM_CORES


def _shuffle(x2, perm):
    mesh = plsc.VectorSubcoreMesh(core_axis_name="c", subcore_axis_name="s")

    @functools.partial(
        pl.kernel,
        mesh=mesh,
        out_type=jax.ShapeDtypeStruct((P, N), jnp.float32),
        scratch_types=[
            pltpu.VMEM((CHUNK,), jnp.int32),
            pltpu.VMEM((CHUNK,), jnp.float32),
            pltpu.VMEM((CHUNK,), jnp.float32),
            pltpu.VMEM_SHARED((N,), jnp.float32),
            pltpu.VMEM_SHARED((N,), jnp.float32),
            pltpu.VMEM_SHARED((N,), jnp.float32),
            pltpu.SemaphoreType.DMA,
            pltpu.SemaphoreType.DMA,
            pltpu.SemaphoreType.DMA,
        ],
    )
    def k(x_hbm, perm_hbm, out_hbm, idx_v, out0, out1, sh0, sh1, sh2,
          sem_l, sem_g, sem_s):
        cid = lax.axis_index("c")
        sid = lax.axis_index("s")
        col0 = sid * CHUNK
        base = cid * PP
        pltpu.sync_copy(perm_hbm.at[pl.ds(col0, CHUNK)], idx_v)

        outs = (out0, out1)

        def load(p, sh):
            return pltpu.make_async_copy(
                x_hbm.at[p, pl.ds(col0, CHUNK)],
                sh.at[pl.ds(col0, CHUNK)],
                sem_l,
            )

        def gather(sh, out_v):
            return pltpu.make_async_copy(sh.at[idx_v], out_v, sem_g)

        def store(p, out_v):
            return pltpu.make_async_copy(
                out_v, out_hbm.at[p, pl.ds(col0, CHUNK)], sem_s
            )

        load(base, sh0).start()
        load(base + 1, sh1).start()
        load(base, sh0).wait()
        plsc.subcore_barrier()
        gather(sh0, out0).start()

        def step(i, sh_cur, sh_nxt, sh_nxt2, out_cur, out_nxt):
            del sh_cur
            @pl.when(i + 1 < base + PP)
            def _():
                load(i + 1, sh_nxt).wait()

            @pl.when(i >= base + 1)
            def _():
                store(i - 1, out_nxt).wait()

            plsc.subcore_barrier()

            @pl.when(i + 1 < base + PP)
            def _():
                gather(sh_nxt, out_nxt).start()

            @pl.when(i + 2 < base + PP)
            def _():
                load(i + 2, sh_nxt2).start()

            pltpu.make_async_copy(sh_nxt.at[idx_v], out_cur, sem_g).wait()
            store(i, out_cur).start()

        def body(j, _):
            i = base + 6 * j
            sh = (sh0, sh1, sh2)
            for u in range(6):
                step(
                    i + u,
                    sh[u % 3],
                    sh[(u + 1) % 3],
                    sh[(u + 2) % 3],
                    outs[u % 2],
                    outs[(u + 1) % 2],
                )
            return ()

        lax.fori_loop(0, PP // 6, body, ())
        store(base, outs[(PP - 1) % 2]).wait()

    return k(x2, perm)


def kernel(x, permutation):
    x2 = x.reshape(P, N)
    perm = permutation.astype(jnp.int32)
    out = _shuffle(x2, perm)
    return out.reshape(x.shape)
